# Initial kernel scaffold; baseline (speedup 1.0000x reference)
import jax
import jax.numpy as jnp
from jax.experimental import pallas as pl


def kernel(x, edge_index, edge_attr, W1, b1, W2, b2, W3, b3, g1, be1, g2, be2, fcW, fcb):
    raise NotImplementedError("write your pallas kernel here")


# speedup vs baseline: 11.0454x; 11.0454x over previous
import functools

import jax
import jax.numpy as jnp
from jax import lax
from jax.experimental import pallas as pl
from jax.experimental.pallas import tpu as pltpu
from jax.experimental.pallas import tpu_sc as plsc

N = 10000
D = 128
H = 64
E = 320000

NC = 2
NS = 16
NW = NC * NS
CH = 128
PC = 80
NCHUNK = NW * PC
E_PAD = NCHUNK * CH
N_PAD = 10240
RPT = N_PAD // NS

_mesh = plsc.VectorSubcoreMesh(core_axis_name="c", subcore_axis_name="s")
_sc_params = pltpu.CompilerParams(use_tc_tiling_on_sc=False)


@functools.partial(
    pl.kernel,
    out_type=jax.ShapeDtypeStruct((NC, N_PAD, 8), jnp.float32),
    mesh=_mesh,
    compiler_params=_sc_params,
    scratch_types=[
        pltpu.VMEM((PC, CH), jnp.int32),
        pltpu.VMEM((CH, 8), jnp.float32),
        pltpu.VMEM_SHARED((N_PAD, 8), jnp.float32),
    ],
)
def _sc_degree(dst_hbm, ones_hbm, zeros_hbm, out_hbm, dst_v, ones_v, acc):
    cid = lax.axis_index("c")
    sid = lax.axis_index("s")
    gwid = cid * NS + sid
    pltpu.sync_copy(zeros_hbm, acc.at[pl.ds(sid * RPT, RPT)])
    pltpu.sync_copy(dst_hbm.at[pl.ds(gwid * PC, PC)], dst_v)
    pltpu.sync_copy(ones_hbm, ones_v)
    plsc.subcore_barrier()

    def chunk(j, carry):
        pltpu.sync_copy(ones_v, acc.at[dst_v.at[j]], add=True)
        return carry

    lax.fori_loop(0, PC, chunk, 0, unroll=False)
    plsc.subcore_barrier()
    pltpu.sync_copy(acc.at[pl.ds(sid * RPT, RPT)],
                    out_hbm.at[cid, pl.ds(sid * RPT, RPT)])


@functools.partial(
    pl.kernel,
    out_type=jax.ShapeDtypeStruct((NC, N_PAD, H), jnp.float32),
    mesh=_mesh,
    compiler_params=_sc_params,
    scratch_types=[
        pltpu.VMEM((PC, CH), jnp.int32),
        pltpu.VMEM((PC, CH), jnp.int32),
        pltpu.VMEM((CH, H), jnp.float32),
        pltpu.VMEM_SHARED((N_PAD, H), jnp.float32),
        pltpu.SemaphoreType.DMA,
    ],
)
def _sc_scatter(y_hbm, src_hbm, dst_hbm, zeros_hbm, out_hbm,
                src_v, dst_v, rows_v, acc, sem):
    cid = lax.axis_index("c")
    sid = lax.axis_index("s")
    gwid = cid * NS + sid
    pltpu.sync_copy(zeros_hbm, acc.at[pl.ds(sid * RPT, RPT)])
    pltpu.sync_copy(src_hbm.at[pl.ds(gwid * PC, PC)], src_v)
    pltpu.sync_copy(dst_hbm.at[pl.ds(gwid * PC, PC)], dst_v)
    plsc.subcore_barrier()

    def chunk(j, carry):
        pltpu.async_copy(y_hbm.at[src_v.at[j]], rows_v, sem).wait()
        pltpu.sync_copy(rows_v, acc.at[dst_v.at[j]], add=True)
        return carry

    lax.fori_loop(0, PC, chunk, 0, unroll=False)
    plsc.subcore_barrier()
    pltpu.sync_copy(acc.at[pl.ds(sid * RPT, RPT)],
                    out_hbm.at[cid, pl.ds(sid * RPT, RPT)])


def _tc1_body(x_ref, w_ref, cnt_ref, y_ref, dinv_ref):
    deg = cnt_ref[0, :N, 0:1] + cnt_ref[1, :N, 0:1] + 1.0
    dinv = lax.rsqrt(deg)
    xw = jnp.dot(x_ref[...], w_ref[...], preferred_element_type=jnp.float32)
    y_ref[...] = xw * dinv
    dinv_ref[...] = dinv


def _tc_mid_body(acc_ref, yprev_ref, dinv_ref, b_ref, g_ref, be_ref, w_ref,
                 ynext_ref):
    agg = acc_ref[0, :N, :] + acc_ref[1, :N, :] + yprev_ref[...]
    dinv = dinv_ref[...]
    t = agg * dinv + b_ref[...]
    mean = jnp.mean(t, axis=0, keepdims=True)
    c = t - mean
    var = jnp.mean(c * c, axis=0, keepdims=True)
    h = jnp.maximum(c * lax.rsqrt(var + 1e-5) * g_ref[...] + be_ref[...], 0.0)
    ynext_ref[...] = jnp.dot(
        h, w_ref[...], preferred_element_type=jnp.float32) * dinv


def _tc_out_body(acc_ref, yprev_ref, dinv_ref, b_ref, fcw_ref, fcb_ref,
                 out_ref):
    agg = acc_ref[0, :N, :] + acc_ref[1, :N, :] + yprev_ref[...]
    t = agg * dinv_ref[...] + b_ref[...]
    h = jnp.maximum(t, 0.0)
    out_ref[...] = jnp.dot(
        h, fcw_ref[...], preferred_element_type=jnp.float32) + fcb_ref[...]


_tc1 = pl.pallas_call(
    _tc1_body,
    out_shape=(jax.ShapeDtypeStruct((N, H), jnp.float32),
               jax.ShapeDtypeStruct((N, 1), jnp.float32)),
)

_tc_mid = pl.pallas_call(
    _tc_mid_body,
    out_shape=jax.ShapeDtypeStruct((N, H), jnp.float32),
)

_tc_out = pl.pallas_call(
    _tc_out_body,
    out_shape=jax.ShapeDtypeStruct((N, 2), jnp.float32),
)


def kernel(x, edge_index, edge_attr, W1, b1, W2, b2, W3, b3, g1, be1, g2, be2,
           fcW, fcb):
    ei = edge_index.astype(jnp.int32)
    pad = E_PAD - E
    src2d = jnp.concatenate(
        [ei[0], jnp.zeros((pad,), jnp.int32)]).reshape(NCHUNK, CH)
    dst2d = jnp.concatenate(
        [ei[1], jnp.full((pad,), N, jnp.int32)]).reshape(NCHUNK, CH)

    ones1 = jnp.ones((CH, 8), jnp.float32)
    zeros1 = jnp.zeros((RPT, 8), jnp.float32)
    zerosH = jnp.zeros((RPT, H), jnp.float32)

    cnt = _sc_degree(dst2d, ones1, zeros1)

    y1, dinv = _tc1(x, W1, cnt)
    acc1 = _sc_scatter(y1, src2d, dst2d, zerosH)
    y2 = _tc_mid(acc1, y1, dinv, b1.reshape(1, H), g1.reshape(1, H),
                 be1.reshape(1, H), W2)
    acc2 = _sc_scatter(y2, src2d, dst2d, zerosH)
    y3 = _tc_mid(acc2, y2, dinv, b2.reshape(1, H), g2.reshape(1, H),
                 be2.reshape(1, H), W3)
    acc3 = _sc_scatter(y3, src2d, dst2d, zerosH)
    return _tc_out(acc3, y3, dinv, b3.reshape(1, H), fcW, fcb.reshape(1, 2))


# speedup vs baseline: 24.2476x; 2.1953x over previous
import functools

import jax
import jax.numpy as jnp
from jax import lax
from jax.experimental import pallas as pl
from jax.experimental.pallas import tpu as pltpu
from jax.experimental.pallas import tpu_sc as plsc

N = 10000
D = 128
H = 64
E = 320000

NC = 2
NS = 16
NW = NC * NS
CH = 128
PC = 80
NCHUNK = NW * PC
E_PAD = NCHUNK * CH
N_PAD = 10240
RPT = N_PAD // NS

_mesh = plsc.VectorSubcoreMesh(core_axis_name="c", subcore_axis_name="s")
_sc_params = pltpu.CompilerParams(use_tc_tiling_on_sc=False)


@functools.partial(
    pl.kernel,
    out_type=jax.ShapeDtypeStruct((NC, N_PAD, 8), jnp.float32),
    mesh=_mesh,
    compiler_params=_sc_params,
    scratch_types=[
        pltpu.VMEM((PC, CH), jnp.int32),
        pltpu.VMEM((CH, 8), jnp.float32),
        pltpu.VMEM_SHARED((N_PAD, 8), jnp.float32),
    ],
)
def _sc_degree(dst_hbm, ones_hbm, zeros_hbm, out_hbm, dst_v, ones_v, acc):
    cid = lax.axis_index("c")
    sid = lax.axis_index("s")
    gwid = cid * NS + sid
    pltpu.sync_copy(zeros_hbm, acc.at[pl.ds(sid * RPT, RPT)])
    pltpu.sync_copy(dst_hbm.at[pl.ds(gwid * PC, PC)], dst_v)
    pltpu.sync_copy(ones_hbm, ones_v)
    plsc.subcore_barrier()

    def chunk(j, carry):
        pltpu.sync_copy(ones_v, acc.at[dst_v.at[j]], add=True)
        return carry

    lax.fori_loop(0, PC, chunk, 0, unroll=False)
    plsc.subcore_barrier()
    pltpu.sync_copy(acc.at[pl.ds(sid * RPT, RPT)],
                    out_hbm.at[cid, pl.ds(sid * RPT, RPT)])


@functools.partial(
    pl.kernel,
    out_type=jax.ShapeDtypeStruct((NC, N_PAD, H), jnp.float32),
    mesh=_mesh,
    compiler_params=_sc_params,
    scratch_types=[
        pltpu.VMEM((PC, CH), jnp.int32),
        pltpu.VMEM((PC, CH), jnp.int32),
        pltpu.VMEM((CH, H), jnp.float32),
        pltpu.VMEM_SHARED((N_PAD, H), jnp.float32),
        pltpu.SemaphoreType.DMA,
    ],
)
def _sc_scatter(y_hbm, src_hbm, dst_hbm, zeros_hbm, out_hbm,
                src_v, dst_v, rows_v, acc, sem):
    cid = lax.axis_index("c")
    sid = lax.axis_index("s")
    gwid = cid * NS + sid
    pltpu.sync_copy(zeros_hbm, acc.at[pl.ds(sid * RPT, RPT)])
    pltpu.sync_copy(src_hbm.at[pl.ds(gwid * PC, PC)], src_v)
    pltpu.sync_copy(dst_hbm.at[pl.ds(gwid * PC, PC)], dst_v)
    plsc.subcore_barrier()

    def chunk(j, carry):
        pltpu.async_copy(y_hbm.at[src_v.at[j]], rows_v, sem).wait()
        pltpu.sync_copy(rows_v, acc.at[dst_v.at[j]], add=True)
        return carry

    lax.fori_loop(0, PC, chunk, 0, unroll=False)
    plsc.subcore_barrier()
    pltpu.sync_copy(acc.at[pl.ds(sid * RPT, RPT)],
                    out_hbm.at[cid, pl.ds(sid * RPT, RPT)])


def _tc1_body(x_ref, w_ref, cnt_ref, y_ref, dinv_ref):
    deg = cnt_ref[0, :N, 0:1] + cnt_ref[1, :N, 0:1] + 1.0
    dinv = lax.rsqrt(deg)
    xw = jnp.dot(x_ref[...], w_ref[...], preferred_element_type=jnp.float32)
    y_ref[...] = xw * dinv
    dinv_ref[...] = dinv


def _tc_mid_body(acc_ref, yprev_ref, dinv_ref, b_ref, g_ref, be_ref, w_ref,
                 ynext_ref):
    agg = acc_ref[0, :N, :] + acc_ref[1, :N, :] + yprev_ref[...]
    dinv = dinv_ref[...]
    t = agg * dinv + b_ref[...]
    mean = jnp.mean(t, axis=0, keepdims=True)
    c = t - mean
    var = jnp.mean(c * c, axis=0, keepdims=True)
    h = jnp.maximum(c * lax.rsqrt(var + 1e-5) * g_ref[...] + be_ref[...], 0.0)
    ynext_ref[...] = jnp.dot(
        h, w_ref[...], preferred_element_type=jnp.float32) * dinv


def _tc_out_body(acc_ref, yprev_ref, dinv_ref, b_ref, fcw_ref, fcb_ref,
                 out_ref):
    agg = acc_ref[0, :N, :] + acc_ref[1, :N, :] + yprev_ref[...]
    t = agg * dinv_ref[...] + b_ref[...]
    h = jnp.maximum(t, 0.0)
    out_ref[...] = jnp.dot(
        h, fcw_ref[...], preferred_element_type=jnp.float32) + fcb_ref[...]


_tc1 = pl.pallas_call(
    _tc1_body,
    out_shape=(jax.ShapeDtypeStruct((N, H), jnp.float32),
               jax.ShapeDtypeStruct((N, 1), jnp.float32)),
)

_tc_mid = pl.pallas_call(
    _tc_mid_body,
    out_shape=jax.ShapeDtypeStruct((N, H), jnp.float32),
)

_tc_out = pl.pallas_call(
    _tc_out_body,
    out_shape=jax.ShapeDtypeStruct((N, 2), jnp.float32),
)


def kernel(x, edge_index, edge_attr, W1, b1, W2, b2, W3, b3, g1, be1, g2, be2,
           fcW, fcb):
    ei = edge_index.astype(jnp.int32)
    pad = E_PAD - E
    pad_src = jnp.arange(pad, dtype=jnp.int32) % N
    pad_dst = N + jnp.arange(pad, dtype=jnp.int32) % (N_PAD - N)
    src2d = jnp.concatenate([ei[0], pad_src]).reshape(NCHUNK, CH)
    dst2d = jnp.concatenate([ei[1], pad_dst]).reshape(NCHUNK, CH)

    ones1 = jnp.ones((CH, 8), jnp.float32)
    zeros1 = jnp.zeros((RPT, 8), jnp.float32)
    zerosH = jnp.zeros((RPT, H), jnp.float32)

    cnt = _sc_degree(dst2d, ones1, zeros1)

    y1, dinv = _tc1(x, W1, cnt)
    acc1 = _sc_scatter(y1, src2d, dst2d, zerosH)
    y2 = _tc_mid(acc1, y1, dinv, b1.reshape(1, H), g1.reshape(1, H),
                 be1.reshape(1, H), W2)
    acc2 = _sc_scatter(y2, src2d, dst2d, zerosH)
    y3 = _tc_mid(acc2, y2, dinv, b2.reshape(1, H), g2.reshape(1, H),
                 be2.reshape(1, H), W3)
    acc3 = _sc_scatter(y3, src2d, dst2d, zerosH)
    return _tc_out(acc3, y3, dinv, b3.reshape(1, H), fcW, fcb.reshape(1, 2))


# speedup vs baseline: 39.4927x; 1.6287x over previous
import functools

import jax
import jax.numpy as jnp
from jax import lax
from jax.experimental import pallas as pl
from jax.experimental.pallas import tpu as pltpu
from jax.experimental.pallas import tpu_sc as plsc

N = 10000
D = 128
H = 64
E = 320000

NC = 2
NS = 16
NW = NC * NS
CH = 128
PC = 80
NCHUNK = NW * PC
E_PAD = NCHUNK * CH
N_PAD = 10240
RPT = N_PAD // NS

_mesh = plsc.VectorSubcoreMesh(core_axis_name="c", subcore_axis_name="s")
_sc_params = pltpu.CompilerParams(use_tc_tiling_on_sc=False)


@functools.partial(
    pl.kernel,
    out_type=jax.ShapeDtypeStruct((NC, N_PAD, 8), jnp.float32),
    mesh=_mesh,
    compiler_params=_sc_params,
    scratch_types=[
        pltpu.VMEM((PC, CH), jnp.int32),
        pltpu.VMEM((CH, 8), jnp.float32),
        pltpu.VMEM_SHARED((N_PAD, 8), jnp.float32),
    ],
)
def _sc_degree(dst_hbm, ones_hbm, zeros_hbm, out_hbm, dst_v, ones_v, acc):
    cid = lax.axis_index("c")
    sid = lax.axis_index("s")
    gwid = cid * NS + sid
    pltpu.sync_copy(zeros_hbm, acc.at[pl.ds(sid * RPT, RPT)])
    pltpu.sync_copy(dst_hbm.at[pl.ds(gwid * PC, PC)], dst_v)
    pltpu.sync_copy(ones_hbm, ones_v)
    plsc.subcore_barrier()

    def chunk(j, carry):
        pltpu.sync_copy(ones_v, acc.at[dst_v.at[j]], add=True)
        return carry

    lax.fori_loop(0, PC, chunk, 0, unroll=False)
    plsc.subcore_barrier()
    pltpu.sync_copy(acc.at[pl.ds(sid * RPT, RPT)],
                    out_hbm.at[cid, pl.ds(sid * RPT, RPT)])


NB = 4
NG = PC // NB


@functools.partial(
    pl.kernel,
    out_type=jax.ShapeDtypeStruct((NC, N_PAD, H), jnp.float32),
    mesh=_mesh,
    compiler_params=_sc_params,
    scratch_types=[
        pltpu.VMEM((PC, CH), jnp.int32),
        pltpu.VMEM((PC, CH), jnp.int32),
        [pltpu.VMEM((CH, H), jnp.float32)] * NB,
        pltpu.VMEM_SHARED((N_PAD, H), jnp.float32),
        [pltpu.SemaphoreType.DMA] * NB,
    ],
)
def _sc_scatter(y_hbm, src_hbm, dst_hbm, zeros_hbm, out_hbm,
                src_v, dst_v, rows, acc, sems):
    cid = lax.axis_index("c")
    sid = lax.axis_index("s")
    gwid = cid * NS + sid
    pltpu.sync_copy(zeros_hbm, acc.at[pl.ds(sid * RPT, RPT)])
    pltpu.sync_copy(src_hbm.at[pl.ds(gwid * PC, PC)], src_v)
    pltpu.sync_copy(dst_hbm.at[pl.ds(gwid * PC, PC)], dst_v)
    plsc.subcore_barrier()

    for b in range(NB):
        pltpu.async_copy(y_hbm.at[src_v.at[b]], rows[b], sems[b])

    def group(g, carry):
        for b in range(NB):
            j = g * NB + b
            pltpu.make_async_copy(y_hbm.at[src_v.at[j]], rows[b],
                                  sems[b]).wait()
            pltpu.sync_copy(rows[b], acc.at[dst_v.at[j]], add=True)

            @pl.when(g < NG - 1)
            def _prefetch(b=b, j=j):
                pltpu.async_copy(y_hbm.at[src_v.at[j + NB]], rows[b], sems[b])
        return carry

    lax.fori_loop(0, NG, group, 0, unroll=False)
    plsc.subcore_barrier()
    pltpu.sync_copy(acc.at[pl.ds(sid * RPT, RPT)],
                    out_hbm.at[cid, pl.ds(sid * RPT, RPT)])


def _tc1_body(x_ref, w_ref, cnt_ref, y_ref, dinv_ref):
    deg = cnt_ref[0, :N, 0:1] + cnt_ref[1, :N, 0:1] + 1.0
    dinv = lax.rsqrt(deg)
    xw = jnp.dot(x_ref[...], w_ref[...], preferred_element_type=jnp.float32)
    y_ref[...] = xw * dinv
    dinv_ref[...] = dinv


def _tc_mid_body(acc_ref, yprev_ref, dinv_ref, b_ref, g_ref, be_ref, w_ref,
                 ynext_ref):
    agg = acc_ref[0, :N, :] + acc_ref[1, :N, :] + yprev_ref[...]
    dinv = dinv_ref[...]
    t = agg * dinv + b_ref[...]
    mean = jnp.mean(t, axis=0, keepdims=True)
    c = t - mean
    var = jnp.mean(c * c, axis=0, keepdims=True)
    h = jnp.maximum(c * lax.rsqrt(var + 1e-5) * g_ref[...] + be_ref[...], 0.0)
    ynext_ref[...] = jnp.dot(
        h, w_ref[...], preferred_element_type=jnp.float32) * dinv


def _tc_out_body(acc_ref, yprev_ref, dinv_ref, b_ref, fcw_ref, fcb_ref,
                 out_ref):
    agg = acc_ref[0, :N, :] + acc_ref[1, :N, :] + yprev_ref[...]
    t = agg * dinv_ref[...] + b_ref[...]
    h = jnp.maximum(t, 0.0)
    out_ref[...] = jnp.dot(
        h, fcw_ref[...], preferred_element_type=jnp.float32) + fcb_ref[...]


_tc1 = pl.pallas_call(
    _tc1_body,
    out_shape=(jax.ShapeDtypeStruct((N, H), jnp.float32),
               jax.ShapeDtypeStruct((N, 1), jnp.float32)),
)

_tc_mid = pl.pallas_call(
    _tc_mid_body,
    out_shape=jax.ShapeDtypeStruct((N, H), jnp.float32),
)

_tc_out = pl.pallas_call(
    _tc_out_body,
    out_shape=jax.ShapeDtypeStruct((N, 2), jnp.float32),
)


def kernel(x, edge_index, edge_attr, W1, b1, W2, b2, W3, b3, g1, be1, g2, be2,
           fcW, fcb):
    ei = edge_index.astype(jnp.int32)
    pad = E_PAD - E
    pad_src = jnp.arange(pad, dtype=jnp.int32) % N
    pad_dst = N + jnp.arange(pad, dtype=jnp.int32) % (N_PAD - N)
    src2d = jnp.concatenate([ei[0], pad_src]).reshape(NCHUNK, CH)
    dst2d = jnp.concatenate([ei[1], pad_dst]).reshape(NCHUNK, CH)

    ones1 = jnp.ones((CH, 8), jnp.float32)
    zeros1 = jnp.zeros((RPT, 8), jnp.float32)
    zerosH = jnp.zeros((RPT, H), jnp.float32)

    cnt = _sc_degree(dst2d, ones1, zeros1)

    y1, dinv = _tc1(x, W1, cnt)
    acc1 = _sc_scatter(y1, src2d, dst2d, zerosH)
    y2 = _tc_mid(acc1, y1, dinv, b1.reshape(1, H), g1.reshape(1, H),
                 be1.reshape(1, H), W2)
    acc2 = _sc_scatter(y2, src2d, dst2d, zerosH)
    y3 = _tc_mid(acc2, y2, dinv, b2.reshape(1, H), g2.reshape(1, H),
                 be2.reshape(1, H), W3)
    acc3 = _sc_scatter(y3, src2d, dst2d, zerosH)
    return _tc_out(acc3, y3, dinv, b3.reshape(1, H), fcW, fcb.reshape(1, 2))
